# baseline (speedup 1.0000x reference)
import functools

import jax
import jax.numpy as jnp
from jax import lax
from jax.experimental import pallas as pl
from jax.experimental.pallas import tpu as pltpu, tpu_sc as plsc

_N = 4194304
_G = 16
_L = 16

_NC = 2
_NS = 16
_NW = _NC * _NS

_PER_W = _N // _NW
_CHUNK = 16384
_NCHUNKS = _PER_W // _CHUNK


def _sc_body(x_hbm, g_hbm, c_hbm, s_hbm, out_hbm,
             x_v, g_v, o_v, c_v, inv_v, sem):
    wid = lax.axis_index("s") * _NC + lax.axis_index("c")
    base = wid * _PER_W

    pltpu.sync_copy(c_hbm, c_v)
    pltpu.sync_copy(s_hbm, inv_v)
    inv_v[...] = 1.0 / inv_v[...]

    def chunk_body(ci, _):
        off = base + ci * _CHUNK
        pltpu.sync_copy(x_hbm.at[pl.ds(off, _CHUNK)], x_v)
        pltpu.sync_copy(g_hbm.at[pl.ds(off, _CHUNK)], g_v)

        def vec_body(i, _):
            sl = pl.ds(i * _L, _L)
            gidx = g_v[sl] - 1
            c = plsc.load_gather(c_v, [gidx])
            a = plsc.load_gather(inv_v, [gidx])
            o_v[sl] = (x_v[sl] - c) * a
            return 0

        lax.fori_loop(0, _CHUNK // _L, vec_body, 0, unroll=4)
        pltpu.sync_copy(o_v, out_hbm.at[pl.ds(off, _CHUNK)])
        return 0

    lax.fori_loop(0, _NCHUNKS, chunk_body, 0)


@jax.jit
def _standardize(x, group, centers, scales):
    mesh = plsc.VectorSubcoreMesh(core_axis_name="c", subcore_axis_name="s")
    return pl.kernel(
        _sc_body,
        out_type=jax.ShapeDtypeStruct((_N,), jnp.float32),
        mesh=mesh,
        scratch_types=[
            pltpu.VMEM((_CHUNK,), jnp.float32),
            pltpu.VMEM((_CHUNK,), jnp.int32),
            pltpu.VMEM((_CHUNK,), jnp.float32),
            pltpu.VMEM((_G,), jnp.float32),
            pltpu.VMEM((_G,), jnp.float32),
            pltpu.SemaphoreType.DMA,
        ],
        compiler_params=pltpu.CompilerParams(needs_layout_passes=False),
    )(x, group, centers, scales)


def kernel(x, group, centers, scales):
    return _standardize(x, group, centers, scales)


# speedup vs baseline: 3.3562x; 3.3562x over previous
import functools

import jax
import jax.numpy as jnp
from jax import lax
from jax.experimental import pallas as pl
from jax.experimental.pallas import tpu as pltpu, tpu_sc as plsc

_N = 4194304
_G = 16
_L = 16

_NC = 2
_NS = 16
_NW = _NC * _NS

_PER_W = _N // _NW
_CHUNK = 16384
_NCHUNKS = _PER_W // _CHUNK
_NBUF = 2

_GATHER_DNUMS = lax.GatherDimensionNumbers(
    offset_dims=(), collapsed_slice_dims=(0,), start_index_map=(0,))


def _vreg_gather(table, idx):
    return lax.gather(table, idx[:, None], _GATHER_DNUMS, (1,),
                      mode=lax.GatherScatterMode.PROMISE_IN_BOUNDS)


def _sc_body(x_hbm, g_hbm, c_hbm, s_hbm, out_hbm,
             x_v, g_v, o_v, c_v, s_v,
             sem_in0, sem_in1, sem_out0, sem_out1):
    wid = lax.axis_index("s") * _NC + lax.axis_index("c")
    base = wid * _PER_W

    sem_in = (sem_in0, sem_in1)
    sem_out = (sem_out0, sem_out1)

    pltpu.sync_copy(c_hbm, c_v)
    pltpu.sync_copy(s_hbm, s_v)
    c_reg = c_v[...]
    a_reg = 1.0 / s_v[...]

    def start_in(ci):
        b = ci % _NBUF
        off = base + ci * _CHUNK
        hx = pltpu.async_copy(x_hbm.at[pl.ds(off, _CHUNK)], x_v[b], sem_in[b])
        hg = pltpu.async_copy(g_hbm.at[pl.ds(off, _CHUNK)], g_v[b], sem_in[b])
        return (hx, hg)

    def start_out(ci):
        b = ci % _NBUF
        off = base + ci * _CHUNK
        return pltpu.async_copy(o_v[b], out_hbm.at[pl.ds(off, _CHUNK)],
                                sem_out[b])

    def compute(ci):
        b = ci % _NBUF
        xb, gb, ob = x_v[b], g_v[b], o_v[b]

        @plsc.parallel_loop(0, _CHUNK, step=_L, unroll=8)
        def _body(i):
            sl = pl.ds(i, _L)
            gidx = gb[sl] - 1
            c = _vreg_gather(c_reg, gidx)
            a = _vreg_gather(a_reg, gidx)
            ob[sl] = (xb[sl] - c) * a

    in_h = {}
    out_h = {}
    for ci in range(min(_NBUF, _NCHUNKS)):
        in_h[ci] = start_in(ci)
    for ci in range(_NCHUNKS):
        for h in in_h.pop(ci):
            h.wait()
        if ci - _NBUF in out_h:
            out_h.pop(ci - _NBUF).wait()
        compute(ci)
        out_h[ci] = start_out(ci)
        if ci + _NBUF < _NCHUNKS:
            in_h[ci + _NBUF] = start_in(ci + _NBUF)
    for ci in sorted(out_h):
        out_h.pop(ci).wait()


@jax.jit
def _standardize(x, group, centers, scales):
    mesh = plsc.VectorSubcoreMesh(core_axis_name="c", subcore_axis_name="s")
    buf = lambda dt: [pltpu.VMEM((_CHUNK,), dt) for _ in range(_NBUF)]
    return pl.kernel(
        _sc_body,
        out_type=jax.ShapeDtypeStruct((_N,), jnp.float32),
        mesh=mesh,
        scratch_types=[
            buf(jnp.float32),
            buf(jnp.int32),
            buf(jnp.float32),
            pltpu.VMEM((_G,), jnp.float32),
            pltpu.VMEM((_G,), jnp.float32),
            pltpu.SemaphoreType.DMA,
            pltpu.SemaphoreType.DMA,
            pltpu.SemaphoreType.DMA,
            pltpu.SemaphoreType.DMA,
        ],
        compiler_params=pltpu.CompilerParams(needs_layout_passes=False),
    )(x, group, centers, scales)


def kernel(x, group, centers, scales):
    return _standardize(x, group, centers, scales)
